# baseline (speedup 1.0000x reference)
import functools

import jax
import jax.numpy as jnp
from jax import lax
from jax.experimental import pallas as pl
from jax.experimental.pallas import tpu as pltpu
from jax.experimental.pallas import tpu_sc as plsc

_ROWS = 16384
_COLS = 200
_DIM = 32
_L = 16
_NW = 32
_STRIPE = _ROWS // _NW


def _sc_body(idx_hbm, t0_hbm, dt_hbm, out_hbm, idx_v, w_v, t0_v, dt_v, o_v, sem):
    wid = lax.axis_index("s") * 2 + lax.axis_index("c")
    base = wid * _STRIPE
    pltpu.sync_copy(t0_hbm, t0_v)
    pltpu.sync_copy(dt_hbm, dt_v)

    @pl.loop(0, _COLS)
    def _(j):
        pltpu.sync_copy(idx_hbm.at[j, pl.ds(base, _STRIPE)], idx_v)
        for c in range(_STRIPE // _L):
            sl = pl.ds(c * _L, _L)
            w_v[sl] = idx_v[sl].astype(jnp.float32)
        for k in range(_DIM):
            t0 = t0_v[k]
            dt = dt_v[k]
            for c in range(_STRIPE // _L):
                sl = pl.ds(c * _L, _L)
                o_v[k, sl] = t0 + w_v[sl] * dt
        pltpu.async_copy(o_v, out_hbm.at[j, :, pl.ds(base, _STRIPE)], sem).wait()


def kernel(inputs, table):
    rows, cols = inputs.shape
    dim = table.shape[1]
    idx_t = inputs.T
    t0_rep = jnp.broadcast_to(table[0].reshape(dim, 1), (dim, _L))
    dt_rep = jnp.broadcast_to((table[1] - table[0]).reshape(dim, 1), (dim, _L))
    mesh = plsc.VectorSubcoreMesh(core_axis_name="c", subcore_axis_name="s")
    sck = pl.kernel(
        _sc_body,
        out_type=jax.ShapeDtypeStruct((cols, dim, rows), jnp.float32),
        mesh=mesh,
        scratch_types=[
            pltpu.VMEM((_STRIPE,), jnp.int32),
            pltpu.VMEM((_STRIPE,), jnp.float32),
            pltpu.VMEM((dim, _L), jnp.float32),
            pltpu.VMEM((dim, _L), jnp.float32),
            pltpu.VMEM((dim, _STRIPE), jnp.float32),
            pltpu.SemaphoreType.DMA,
        ],
        compiler_params=pltpu.CompilerParams(use_tc_tiling_on_sc=True),
    )
    out_t = sck(idx_t, t0_rep, dt_rep)
    return out_t.transpose(2, 0, 1)


# speedup vs baseline: 1.1536x; 1.1536x over previous
import functools

import jax
import jax.numpy as jnp
from jax import lax
from jax.experimental import pallas as pl
from jax.experimental.pallas import tpu as pltpu
from jax.experimental.pallas import tpu_sc as plsc

_ROWS = 16384
_COLS = 200
_DIM = 32
_L = 16
_NW = 32
_STRIPE = _ROWS // _NW


def _sc_body(idx_hbm, t0_hbm, dt_hbm, out_hbm,
             idx_v, w_v, t0_v, dt_v, o_v, isem0, isem1, osem0, osem1):
    wid = lax.axis_index("s") * 2 + lax.axis_index("c")
    base = wid * _STRIPE
    pltpu.sync_copy(t0_hbm, t0_v)
    pltpu.sync_copy(dt_hbm, dt_v)
    isems = (isem0, isem1)
    osems = (osem0, osem1)

    def idx_copy(j, b):
        return pltpu.make_async_copy(
            idx_hbm.at[j, pl.ds(base, _STRIPE)], idx_v.at[b], isems[b])

    def out_copy(j, b):
        return pltpu.make_async_copy(
            o_v.at[b], out_hbm.at[j, :, pl.ds(base, _STRIPE)], osems[b])

    idx_copy(0, 0).start()
    idx_copy(1, 1).start()

    @pl.loop(0, _COLS // 2)
    def _(jj):
        for b in range(2):
            j = jj * 2 + b
            idx_copy(j, b).wait()
            for c in range(_STRIPE // _L):
                sl = pl.ds(c * _L, _L)
                w_v[sl] = idx_v[b, sl].astype(jnp.float32)

            @pl.when(j + 2 < _COLS)
            def _():
                idx_copy(j + 2, b).start()

            @pl.when(jj > 0)
            def _():
                out_copy(j - 2, b).wait()

            for k in range(_DIM):
                t0 = t0_v[k]
                dt = dt_v[k]
                for c in range(_STRIPE // _L):
                    sl = pl.ds(c * _L, _L)
                    o_v[b, k, sl] = t0 + w_v[sl] * dt
            out_copy(j, b).start()

    out_copy(_COLS - 2, 0).wait()
    out_copy(_COLS - 1, 1).wait()


def kernel(inputs, table):
    rows, cols = inputs.shape
    dim = table.shape[1]
    idx_t = inputs.T
    t0_rep = jnp.broadcast_to(table[0].reshape(dim, 1), (dim, _L))
    dt_rep = jnp.broadcast_to((table[1] - table[0]).reshape(dim, 1), (dim, _L))
    mesh = plsc.VectorSubcoreMesh(core_axis_name="c", subcore_axis_name="s")
    sck = pl.kernel(
        _sc_body,
        out_type=jax.ShapeDtypeStruct((cols, dim, rows), jnp.float32),
        mesh=mesh,
        scratch_types=[
            pltpu.VMEM((2, _STRIPE), jnp.int32),
            pltpu.VMEM((_STRIPE,), jnp.float32),
            pltpu.VMEM((dim, _L), jnp.float32),
            pltpu.VMEM((dim, _L), jnp.float32),
            pltpu.VMEM((2, dim, _STRIPE), jnp.float32),
            pltpu.SemaphoreType.DMA,
            pltpu.SemaphoreType.DMA,
            pltpu.SemaphoreType.DMA,
            pltpu.SemaphoreType.DMA,
        ],
        compiler_params=pltpu.CompilerParams(use_tc_tiling_on_sc=True),
    )
    out_t = sck(idx_t, t0_rep, dt_rep)
    return out_t.transpose(2, 0, 1)
